# baseline (speedup 1.0000x reference)
import functools

import jax
import jax.numpy as jnp
import numpy as np
from jax import lax
from jax.experimental import pallas as pl
from jax.experimental.pallas import tpu as pltpu
from jax.experimental.pallas import tpu_sc as plsc

_NUM_CORES = 2
_NUM_SUBCORES = 16
_NW = _NUM_CORES * _NUM_SUBCORES

_LANES = 16
_ROWS = 256
_TCOLS = _ROWS // 128


def _perm_table(d):
    n_units = _ROWS * d // _LANES
    perm = np.empty((n_units, _LANES), np.int32)
    for j in range(n_units):
        r = (j * _LANES) // d
        f0 = (j * _LANES) % d
        for i in range(_LANES):
            f = f0 + i
            perm[j, i] = ((f // 8) * _TCOLS + r // 128) * 1024 + (f % 8) * 128 + (r % 128)
    return perm


@functools.partial(jax.jit, static_argnums=(3, 4))
def _gather_tiled(idx, table, perm, n, d):
    per_w = n // _NW
    n_chunks = per_w // _ROWS
    n_units = _ROWS * d // _LANES
    cols_per_w = per_w // 128
    obuf = _TCOLS * 1024

    @functools.partial(
        pl.kernel,
        out_type=jax.ShapeDtypeStruct((n * d,), jnp.float32),
        mesh=plsc.VectorSubcoreMesh(core_axis_name="c", subcore_axis_name="s"),
        scratch_types=[
            pltpu.VMEM((per_w,), jnp.int32),
            pltpu.VMEM((2, _ROWS, d), jnp.float32),
            pltpu.VMEM((2, 4 * obuf), jnp.float32),
            pltpu.VMEM((n_units, _LANES), jnp.int32),
            pltpu.SemaphoreType.DMA((2,)),
            pltpu.SemaphoreType.DMA((2,)),
        ],
        compiler_params=pltpu.CompilerParams(
            use_tc_tiling_on_sc=False, needs_layout_passes=False),
    )
    def _impl(idx_hbm, table_hbm, perm_hbm, out_hbm, idx_v, rbuf, tbuf, perm_v,
              gsem, wsem):
        wid = lax.axis_index("s") * _NUM_CORES + lax.axis_index("c")
        base = wid * per_w
        tcol0 = wid * cols_per_w

        pltpu.sync_copy(idx_hbm.at[pl.ds(base, per_w)], idx_v)
        pltpu.sync_copy(perm_hbm, perm_v)

        def gather_of(c, b):
            return pltpu.make_async_copy(
                table_hbm.at[idx_v.at[pl.ds(c * _ROWS, _ROWS)]],
                rbuf.at[b],
                gsem.at[b],
            )

        def write_of(c, b, ft):
            t0 = tcol0 + c * _TCOLS
            l0 = t0 // 128
            bt0 = lax.rem(t0, 128)
            off = ((l0 * 4 + ft) * 128 + bt0) * 1024
            return pltpu.make_async_copy(
                tbuf.at[b, pl.ds(ft * obuf, obuf)],
                out_hbm.at[pl.ds(off, obuf)],
                wsem.at[b],
            )

        def do_sub(c, b):
            gather_of(c, b).wait()

            @pl.when(c >= 2)
            def _():
                for ft in range(4):
                    write_of(c - 2, b, ft).wait()

            tflat = tbuf.at[b]
            for j in range(n_units):
                r = (j * _LANES) // d
                h = (j * _LANES) % d
                v = rbuf[b, r, pl.ds(h, _LANES)]
                plsc.store_scatter(tflat, [perm_v[j]], v)
            for ft in range(4):
                write_of(c, b, ft).start()

            @pl.when(c + 2 < n_chunks)
            def _():
                gather_of(c + 2, b).start()

        gather_of(0, 0).start()
        gather_of(1, 1).start()

        def body(k, carry):
            do_sub(2 * k, 0)
            do_sub(2 * k + 1, 1)
            return carry

        lax.fori_loop(0, n_chunks // 2, body, 0)

        for c in (n_chunks - 2, n_chunks - 1):
            for ft in range(4):
                write_of(c, c % 2, ft).wait()

    return _impl(idx, table, perm)


def kernel(x, table):
    b, l = x.shape
    v, d = table.shape
    n = b * l
    perm = jnp.asarray(_perm_table(d))
    out = _gather_tiled(x.T.reshape(n), table, perm, n, d)
    o5 = out.reshape(l, 4, b // 128, 8, 128)
    return o5.transpose((2, 4, 0, 1, 3)).reshape(b, l, d)


# speedup vs baseline: 1.5489x; 1.5489x over previous
import functools

import jax
import jax.numpy as jnp
from jax import lax
from jax.experimental import pallas as pl
from jax.experimental.pallas import tpu as pltpu
from jax.experimental.pallas import tpu_sc as plsc

_NUM_CORES = 2
_NUM_SUBCORES = 16
_NW = _NUM_CORES * _NUM_SUBCORES

_LANES = 16
_ROWS = 256
_TCOLS = _ROWS // 128
_OBUF = _TCOLS * 1024


@functools.partial(jax.jit, static_argnums=(2, 3))
def _gather_tiled(idx, table, n, d):
    per_w = n // _NW
    n_chunks = per_w // _ROWS
    cols_per_w = per_w // 128

    @functools.partial(
        pl.kernel,
        out_type=jax.ShapeDtypeStruct((n * d,), jnp.float32),
        mesh=plsc.VectorSubcoreMesh(core_axis_name="c", subcore_axis_name="s"),
        scratch_types=[
            pltpu.VMEM((per_w,), jnp.int32),
            pltpu.VMEM((2, _ROWS, d), jnp.float32),
            pltpu.VMEM((2, 4 * _OBUF), jnp.float32),
            pltpu.SemaphoreType.DMA((2,)),
            pltpu.SemaphoreType.DMA((2,)),
        ],
        compiler_params=pltpu.CompilerParams(
            use_tc_tiling_on_sc=False, needs_layout_passes=False),
    )
    def _impl(idx_hbm, table_hbm, out_hbm, idx_v, rbuf, tbuf, gsem, wsem):
        wid = lax.axis_index("s") * _NUM_CORES + lax.axis_index("c")
        base = wid * per_w
        tcol0 = wid * cols_per_w

        pltpu.sync_copy(idx_hbm.at[pl.ds(base, per_w)], idx_v)

        iota = lax.iota(jnp.int32, _LANES)
        fpat = [lax.rem(iota + k, _LANES) for k in range(_LANES)]
        dpat = [(fp // 8) * _OBUF + lax.rem(fp, 8) * 128 + iota for fp in fpat]

        def gather_of(c, b):
            return pltpu.make_async_copy(
                table_hbm.at[idx_v.at[pl.ds(c * _ROWS, _ROWS)]],
                rbuf.at[b],
                gsem.at[b],
            )

        def write_of(c, b, ft):
            t0 = tcol0 + c * _TCOLS
            l0 = t0 // 128
            bt0 = lax.rem(t0, 128)
            off = ((l0 * 4 + ft) * 128 + bt0) * 1024
            return pltpu.make_async_copy(
                tbuf.at[b, pl.ds(ft * _OBUF, _OBUF)],
                out_hbm.at[pl.ds(off, _OBUF)],
                wsem.at[b],
            )

        def do_sub(c, b):
            gather_of(c, b).wait()

            @pl.when(c >= 2)
            def _():
                for ft in range(4):
                    write_of(c - 2, b, ft).wait()

            rb = rbuf.at[b]
            tflat = tbuf.at[b]

            def g_body(g, carry):
                r0 = g * _LANES
                rows = iota + r0
                sbase0 = (r0 // 128) * 1024 + lax.rem(r0, 128)
                for f0 in (0, 16):
                    sbase = sbase0 + (f0 // 8) * _OBUF
                    for k in range(_LANES):
                        v = plsc.load_gather(rb, [rows, fpat[k] + f0])
                        plsc.store_scatter(tflat, [dpat[k] + sbase], v)
                return carry

            lax.fori_loop(0, _ROWS // _LANES, g_body, 0)
            for ft in range(4):
                write_of(c, b, ft).start()

            @pl.when(c + 2 < n_chunks)
            def _():
                gather_of(c + 2, b).start()

        gather_of(0, 0).start()
        gather_of(1, 1).start()

        def body(k, carry):
            do_sub(2 * k, 0)
            do_sub(2 * k + 1, 1)
            return carry

        lax.fori_loop(0, n_chunks // 2, body, 0)

        for c in (n_chunks - 2, n_chunks - 1):
            for ft in range(4):
                write_of(c, c % 2, ft).wait()

    return _impl(idx, table)


def kernel(x, table):
    b, l = x.shape
    v, d = table.shape
    n = b * l
    out = _gather_tiled(x.T.reshape(n), table, n, d)
    o5 = out.reshape(l, 4, b // 128, 8, 128)
    return o5.transpose((2, 4, 0, 1, 3)).reshape(b, l, d)
